# baseline (speedup 1.0000x reference)
import functools

import jax
import jax.numpy as jnp
from jax import lax
from jax.experimental import pallas as pl
from jax.experimental.pallas import tpu as pltpu
from jax.experimental.pallas import tpu_sc as plsc

SUM_N = 100000
HID = 128
M = 500000
N_OPS = 1000
RULE_DIM = 32
RPAD = 33
WORDS = RULE_DIM // 2
WPAD = WORDS + 1

L = 16
NW = 32
CHUNK = 800
GSUB = 80
NSUB = CHUNK // GSUB
NCHUNKS = M // CHUNK
TSTEPS = -(-NCHUNKS // NW)
NGROUPS = CHUNK // L

MM_ROWS = 2000


def _mm_body(h_ref, w_ref, o_ref):
    o_ref[...] = jnp.dot(h_ref[...], w_ref[...],
                         preferred_element_type=jnp.float32
                         ).astype(jnp.bfloat16)


def _project(h_nodes, w):
    grid = SUM_N // MM_ROWS
    return pl.pallas_call(
        _mm_body,
        grid=(grid,),
        in_specs=[
            pl.BlockSpec((MM_ROWS, HID), lambda i: (i, 0)),
            pl.BlockSpec((HID, RULE_DIM), lambda i: (0, 0)),
        ],
        out_specs=pl.BlockSpec((MM_ROWS, RULE_DIM), lambda i: (i, 0)),
        out_shape=jax.ShapeDtypeStruct((SUM_N, RULE_DIM), jnp.bfloat16),
    )(h_nodes, w)


def _sc_body(p_hbm, rule_hbm, b_hbm, nodes_hbm, ops_hbm, out_hbm,
             rule_v, nodes_v, ops_v, prow_v, prow17_v, out_v, b_v,
             sem_g0, sem_g1):
    wid = lax.axis_index("s") * 2 + lax.axis_index("c")
    sem_g = (sem_g0, sem_g1)

    pltpu.sync_copy(rule_hbm, rule_v)
    pltpu.sync_copy(b_hbm, b_v)
    bsplat = b_v[...]
    iota = lax.iota(jnp.int32, L)

    def fire(q, cid):
        base = pl.multiple_of(cid * CHUNK, 8)
        pltpu.sync_copy(nodes_hbm.at[pl.ds(base, CHUNK)], nodes_v.at[q])
        pltpu.sync_copy(ops_hbm.at[pl.ds(base, CHUNK)], ops_v.at[q])
        for r in range(NSUB):
            pltpu.async_copy(
                p_hbm.at[nodes_v.at[q, pl.ds(r * GSUB, GSUB)]],
                prow_v.at[q, pl.ds(r * GSUB, GSUB)], sem_g[q])

    def gather_wait(q):
        pltpu.make_async_copy(
            p_hbm.at[pl.ds(0, CHUNK)], prow_v.at[q], sem_g[q]).wait()

    def compute(q):
        prow_q = prow_v.at[q]

        def relay(g, carry):
            for u in range(8):
                c = g * 8 + u
                prow17_v[c, pl.ds(0, WORDS)] = prow_q[c, pl.ds(0, WORDS)]
            return carry

        lax.fori_loop(0, CHUNK // 8, relay, None)
        ops_r, prow_r, out_r = ops_v.at[q], prow17_v, out_v.at[q]

        def group(g, carry):
            rows = iota + g * L
            ops16 = ops_r[pl.ds(g * L, L)]
            acc = bsplat
            for w in range(WORDS):
                widx = jnp.full((L,), w, jnp.int32)
                pw = plsc.load_gather(prow_r, [rows, widx])
                ew = plsc.load_gather(rule_v, [ops16, widx])
                prod = plsc.bitcast(pw, jnp.bfloat16) * plsc.bitcast(
                    ew, jnp.bfloat16)
                lo, hi = plsc.unpack(prod, format=plsc.PackFormat.INTERLEAVED)
                acc = acc + lo + hi
            out_r[pl.ds(g * L, L)] = acc
            return carry

        lax.fori_loop(0, NGROUPS, group, None)

    fire(0, wid)

    def pair(i, carry):
        for k in (0, 1):
            q = k
            cid = wid + (2 * i + k) * NW

            @pl.when(cid < NCHUNKS)
            def _(q=q, cid=cid):
                gather_wait(q)

                @pl.when(cid + NW < NCHUNKS)
                def _():
                    fire(1 - q, cid + NW)

                compute(q)
                pltpu.sync_copy(
                    out_v.at[q],
                    out_hbm.at[pl.ds(pl.multiple_of(cid * CHUNK, 8), CHUNK)])
        return carry

    lax.fori_loop(0, TSTEPS // 2, pair, None)


def _sc_score(p, rule_table, b_bl, nodes, ops):
    mesh = plsc.VectorSubcoreMesh(core_axis_name="c", subcore_axis_name="s")
    fn = pl.kernel(
        _sc_body,
        out_type=jax.ShapeDtypeStruct((M,), jnp.float32),
        mesh=mesh,
        scratch_types=[
            pltpu.VMEM((N_OPS, WPAD), jnp.int32),
            pltpu.VMEM((2, CHUNK), jnp.int32),
            pltpu.VMEM((2, CHUNK), jnp.int32),
            pltpu.VMEM((2, CHUNK, WORDS), jnp.int32),
            pltpu.VMEM((CHUNK, WPAD), jnp.int32),
            pltpu.VMEM((2, CHUNK), jnp.float32),
            pltpu.VMEM((L,), jnp.float32),
            pltpu.SemaphoreType.DMA,
            pltpu.SemaphoreType.DMA,
        ],
        compiler_params=pltpu.CompilerParams(
            needs_layout_passes=False, use_tc_tiling_on_sc=False),
    )
    return fn(p, rule_table, b_bl, nodes, ops)


def _pairs_i32(x):
    n, d = x.shape
    return jax.lax.bitcast_convert_type(
        x.reshape(n, d // 2, 2), jnp.int32)


def kernel(h_nodes, rule_table, W_bl, b_bl, cand_nodes, cand_ops):
    w = W_bl[0]
    p = _pairs_i32(_project(h_nodes, w))
    rule_pad = jnp.pad(_pairs_i32(rule_table.astype(jnp.bfloat16)),
                       ((0, 0), (0, WPAD - WORDS)))
    nodes = cand_nodes.astype(jnp.int32)
    ops = cand_ops.astype(jnp.int32)
    b16 = jnp.broadcast_to(b_bl.astype(jnp.float32), (L,))
    return _sc_score(p, rule_pad, b16, nodes, ops)


# speedup vs baseline: 1.1193x; 1.1193x over previous
import functools

import jax
import jax.numpy as jnp
from jax import lax
from jax.experimental import pallas as pl
from jax.experimental.pallas import tpu as pltpu
from jax.experimental.pallas import tpu_sc as plsc

SUM_N = 100000
HID = 128
M = 500000
N_OPS = 1000
RULE_DIM = 32
RPAD = 33

L = 16
NW = 32
CHUNK = 800
GSUB = 80
NSUB = CHUNK // GSUB
NCHUNKS = M // CHUNK
TSTEPS = -(-NCHUNKS // NW)
NGROUPS = CHUNK // L

MM_ROWS = 2000


def _mm_body(h_ref, w_ref, o_ref):
    o_ref[...] = jnp.dot(h_ref[...], w_ref[...],
                         preferred_element_type=jnp.float32)


def _project(h_nodes, w):
    grid = SUM_N // MM_ROWS
    return pl.pallas_call(
        _mm_body,
        grid=(grid,),
        in_specs=[
            pl.BlockSpec((MM_ROWS, HID), lambda i: (i, 0)),
            pl.BlockSpec((HID, RULE_DIM), lambda i: (0, 0)),
        ],
        out_specs=pl.BlockSpec((MM_ROWS, RULE_DIM), lambda i: (i, 0)),
        out_shape=jax.ShapeDtypeStruct((SUM_N, RULE_DIM), jnp.float32),
    )(h_nodes, w)


def _sc_body(p_hbm, rule_hbm, b_hbm, nodes_hbm, ops_hbm, out_hbm,
             rule_v, nodes_v, ops_v, prow_v, prow33_v, out_v, b_v,
             sem_g0, sem_g1, sem_i0, sem_i1, sem_w0, sem_w1):
    wid = lax.axis_index("s") * 2 + lax.axis_index("c")
    sem_g = (sem_g0, sem_g1)
    sem_i = (sem_i0, sem_i1)
    sem_w = (sem_w0, sem_w1)

    pltpu.sync_copy(rule_hbm, rule_v)
    pltpu.sync_copy(b_hbm, b_v)
    bsplat = b_v[...]
    iota = lax.iota(jnp.int32, L)

    def fire_idx(q, cid):
        base = pl.multiple_of(cid * CHUNK, 8)
        pltpu.async_copy(nodes_hbm.at[pl.ds(base, CHUNK)], nodes_v.at[q],
                         sem_i[q])
        pltpu.async_copy(ops_hbm.at[pl.ds(base, CHUNK)], ops_v.at[q],
                         sem_i[q])

    def idx_wait(q):
        pltpu.make_async_copy(
            nodes_hbm.at[pl.ds(0, CHUNK)], nodes_v.at[q], sem_i[q]).wait()
        pltpu.make_async_copy(
            ops_hbm.at[pl.ds(0, CHUNK)], ops_v.at[q], sem_i[q]).wait()

    def fire_gathers(q):
        for r in range(NSUB):
            pltpu.async_copy(
                p_hbm.at[nodes_v.at[q, pl.ds(r * GSUB, GSUB)]],
                prow_v.at[q, pl.ds(r * GSUB, GSUB)], sem_g[q])

    def gather_wait(q):
        pltpu.make_async_copy(
            p_hbm.at[pl.ds(0, CHUNK)], prow_v.at[q], sem_g[q]).wait()

    def wb_wait(q):
        pltpu.make_async_copy(
            out_hbm.at[pl.ds(0, CHUNK)], out_v.at[q], sem_w[q]).wait()

    def compute(q):
        prow_q = prow_v.at[q]

        def relay(g, carry):
            for u in range(8):
                c = g * 8 + u
                prow33_v[c, pl.ds(0, L)] = prow_q[c, pl.ds(0, L)]
                prow33_v[c, pl.ds(L, L)] = prow_q[c, pl.ds(L, L)]
            return carry

        lax.fori_loop(0, CHUNK // 8, relay, None)
        ops_r, prow_r, out_r = ops_v.at[q], prow33_v, out_v.at[q]

        def group(g, carry):
            rows = iota + g * L
            ops16 = ops_r[pl.ds(g * L, L)]
            acc = bsplat
            for j in range(RULE_DIM):
                jidx = jnp.full((L,), j, jnp.int32)
                pcol = plsc.load_gather(prow_r, [rows, jidx])
                ecol = plsc.load_gather(rule_v, [ops16, jidx])
                acc = acc + pcol * ecol
            out_r[pl.ds(g * L, L)] = acc
            return carry

        lax.fori_loop(0, NGROUPS, group, None)

    fire_idx(0, wid)
    idx_wait(0)
    fire_gathers(0)

    @pl.when(wid + NW < NCHUNKS)
    def _():
        fire_idx(1, wid + NW)

    def pair(i, carry):
        for k in (0, 1):
            q = k
            cid = wid + (2 * i + k) * NW

            @pl.when(cid < NCHUNKS)
            def _(q=q, cid=cid):
                gather_wait(q)

                @pl.when(cid + NW < NCHUNKS)
                def _():
                    idx_wait(1 - q)
                    fire_gathers(1 - q)

                @pl.when(cid >= 2 * NW)
                def _():
                    wb_wait(q)

                compute(q)

                @pl.when(cid + 2 * NW < NCHUNKS)
                def _():
                    fire_idx(q, cid + 2 * NW)

                pltpu.async_copy(
                    out_v.at[q],
                    out_hbm.at[pl.ds(pl.multiple_of(cid * CHUNK, 8), CHUNK)],
                    sem_w[q])
        return carry

    lax.fori_loop(0, TSTEPS // 2, pair, None)

    for t in range(TSTEPS):
        cid = wid + t * NW

        @pl.when((cid < NCHUNKS) & (cid + 2 * NW >= NCHUNKS))
        def _(q=t % 2):
            wb_wait(q)


def _sc_score(p, rule_table, b_bl, nodes, ops):
    mesh = plsc.VectorSubcoreMesh(core_axis_name="c", subcore_axis_name="s")
    fn = pl.kernel(
        _sc_body,
        out_type=jax.ShapeDtypeStruct((M,), jnp.float32),
        mesh=mesh,
        scratch_types=[
            pltpu.VMEM((N_OPS, RPAD), jnp.float32),
            pltpu.VMEM((2, CHUNK), jnp.int32),
            pltpu.VMEM((2, CHUNK), jnp.int32),
            pltpu.VMEM((2, CHUNK, RULE_DIM), jnp.float32),
            pltpu.VMEM((CHUNK, RPAD), jnp.float32),
            pltpu.VMEM((2, CHUNK), jnp.float32),
            pltpu.VMEM((L,), jnp.float32),
            pltpu.SemaphoreType.DMA,
            pltpu.SemaphoreType.DMA,
            pltpu.SemaphoreType.DMA,
            pltpu.SemaphoreType.DMA,
            pltpu.SemaphoreType.DMA,
            pltpu.SemaphoreType.DMA,
        ],
        compiler_params=pltpu.CompilerParams(
            needs_layout_passes=False, use_tc_tiling_on_sc=False),
    )
    return fn(p, rule_table, b_bl, nodes, ops)


def kernel(h_nodes, rule_table, W_bl, b_bl, cand_nodes, cand_ops):
    w = W_bl[0]
    p = _project(h_nodes, w)
    rule_pad = jnp.pad(rule_table, ((0, 0), (0, RPAD - RULE_DIM)))
    nodes = cand_nodes.astype(jnp.int32)
    ops = cand_ops.astype(jnp.int32)
    b16 = jnp.broadcast_to(b_bl.astype(jnp.float32), (L,))
    return _sc_score(p, rule_pad, b16, nodes, ops)
